# baseline (speedup 1.0000x reference)
import functools

import jax
import jax.numpy as jnp
import numpy as np
from jax import lax
from jax.experimental import pallas as pl
from jax.experimental.pallas import tpu as pltpu
from jax.experimental.pallas import tpu_sc as plsc

B = 16384
E = 512
M = 4

B_SC = 3584
B_TC = B - B_SC
SC_WORKERS = 32
SC_CH = 16

_TINY = np.float32(1.17549435e-38)
_ROTS = (13, 15, 26, 6, 17, 29, 16, 24)

_LOG2_POLY = (
    -0.008764015229918067, 0.11976667205066446, -0.7261527889916303,
    2.5703314856108475, -5.882795874749627, 9.127889180021223,
    -9.888683565729947, 8.104570518183051, -3.416161479893353,
)


def _rotl(v, r):
    return lax.shift_left(v, jnp.int32(r)) | lax.shift_right_logical(
        v, jnp.int32(32 - r)
    )


def _threefry2x32_hash(cnt):
    ks = (jnp.int32(0), jnp.int32(42), jnp.int32(0 ^ 42 ^ 0x1BD11BDA))

    x1 = cnt + ks[1]
    x0 = x1
    x1 = x0 ^ _rotl(x1, _ROTS[0])
    for r in _ROTS[1:4]:
        x0 = x0 + x1
        x1 = x0 ^ _rotl(x1, r)
    x0 = x0 + ks[1]
    x1 = x1 + ks[2] + jnp.int32(1)
    for group in range(1, 5):
        rots = _ROTS[0:4] if group % 2 == 0 else _ROTS[4:8]
        for r in rots:
            x0 = x0 + x1
            x1 = x0 ^ _rotl(x1, r)
        x0 = x0 + ks[(group + 1) % 3]
        x1 = x1 + ks[(group + 2) % 3] + jnp.int32(group + 1)
    return x0 ^ x1


def _mantissa_bits(cnt):
    bits = _threefry2x32_hash(cnt)
    return lax.shift_right_logical(bits, jnp.int32(9))


def _tc_log2u(cnt):
    k = _mantissa_bits(cnt)
    fb = k | jnp.int32(0x3F800000)
    fl = lax.bitcast_convert_type(fb, jnp.float32) - jnp.float32(1.0)
    u = fl + _TINY
    return jnp.log2(u)


def _tc_body(m0_ref, m1_ref, m2_ref, m3_ref, av_ref, sp_ref, out_ref, *, tr, cr):
    i = pl.program_id(0)
    refs = (m0_ref, m1_ref, m2_ref, m3_ref)
    row = lax.broadcasted_iota(jnp.int32, (cr, E), 0)
    col = lax.broadcasted_iota(jnp.int32, (cr, E), 1)
    f_local = row * (E * M) + col * M

    def chunk_body(c, carry):
        r0 = c * cr
        f = (i * tr + r0) * (E * M) + f_local
        q = av_ref[pl.ds(r0, cr), :] * sp_ref[pl.ds(r0, cr), :]
        rq = jnp.float32(1.0) / q
        best = _tc_log2u(f) * rq[:, 0:1]
        val = refs[0][pl.ds(r0, cr), :]
        for m in range(1, M):
            s = _tc_log2u(f + m) * rq[:, m : m + 1]
            take = s > best
            val = jnp.where(take, refs[m][pl.ds(r0, cr), :], val)
            best = jnp.where(take, s, best)
        out_ref[pl.ds(r0, cr), :] = val
        return carry

    lax.fori_loop(0, tr // cr, chunk_body, 0, unroll=8)


def _tc_call(m0, m1, m2, m3, availabilities, selection_probabilities):
    tr = 512
    cr = 8
    grid = (B_TC // tr,)
    row_spec = pl.BlockSpec((tr, E), lambda i: (i, 0))
    prob_spec = pl.BlockSpec((tr, M), lambda i: (i, 0))
    return pl.pallas_call(
        functools.partial(_tc_body, tr=tr, cr=cr),
        grid=grid,
        in_specs=[row_spec, row_spec, row_spec, row_spec, prob_spec, prob_spec],
        out_specs=row_spec,
        out_shape=jax.ShapeDtypeStruct((B_TC, E), jnp.float32),
        compiler_params=pltpu.CompilerParams(
            dimension_semantics=("parallel",)
        ),
    )(m0, m1, m2, m3, availabilities, selection_probabilities)


def _sc_log2u(cnt):
    k = _mantissa_bits(cnt)
    g = lax.convert_element_type(k, jnp.float32)
    gb = lax.bitcast_convert_type(g, jnp.int32)
    e = lax.convert_element_type(
        lax.shift_right_logical(gb, jnp.int32(23)) - jnp.int32(127),
        jnp.float32,
    )
    mant = lax.bitcast_convert_type(
        (gb & jnp.int32(0x007FFFFF)) | jnp.int32(0x3F800000), jnp.float32
    )
    p = jnp.float32(_LOG2_POLY[0])
    for c in _LOG2_POLY[1:]:
        p = p * mant + jnp.float32(c)
    return jnp.where(
        k == 0, jnp.float32(-126.0), (e + p) - jnp.float32(23.0)
    )


def _sc_worker(
    m0_hbm, m1_hbm, m2_hbm, m3_hbm, av_hbm, sp_hbm, out_hbm,
    vm0, vm1, vm2, vm3, vav, vsp, vrq, vout, sem,
):
    rows_per_w = B_SC // SC_WORKERS
    wid = lax.axis_index("s") * 2 + lax.axis_index("c")
    row_base = wid * rows_per_w
    vms = (vm0, vm1, vm2, vm3)
    mhbms = (m0_hbm, m1_hbm, m2_hbm, m3_hbm)
    lanes = lax.iota(jnp.int32, 16)

    pltpu.sync_copy(
        av_hbm.at[pl.ds((B_TC + row_base) * M, rows_per_w * M)], vav
    )
    pltpu.sync_copy(
        sp_hbm.at[pl.ds((B_TC + row_base) * M, rows_per_w * M)], vsp
    )

    def chunk_body(ch, carry):
        row0 = row_base + ch * SC_CH
        grow0 = B_TC + row0
        handles = [
            pltpu.async_copy(mh.at[pl.ds(grow0, SC_CH)], vm, sem)
            for vm, mh in zip(vms, mhbms)
        ]

        for r4 in range(SC_CH // 4):
            off4 = ch * (SC_CH * M) + r4 * 16
            q = vav[pl.ds(off4, 16)] * vsp[pl.ds(off4, 16)]
            rq = jnp.float32(1.0) / q
            for j in range(16):
                vrq[pl.ds((r4 * 16 + j) * 16, 16)] = jnp.full(
                    (16,), rq[j], jnp.float32
                )

        for h in handles:
            h.wait()

        def rowcol_body(t, carry2):
            r = t // (E // 16)
            g = lax.rem(t, E // 16)
            off = r * E + g * 16
            f0 = (grow0 + r) * (E * M) + g * (16 * M) + lanes * M
            c0 = g * 16
            best = _sc_log2u(f0) * vrq[pl.ds((r * 4) * 16, 16)]
            val = vm0[r, pl.ds(c0, 16)]
            for m in range(1, M):
                s = _sc_log2u(f0 + m) * vrq[pl.ds((r * 4 + m) * 16, 16)]
                take = s > best
                val = jnp.where(take, vms[m][r, pl.ds(c0, 16)], val)
                best = jnp.where(take, s, best)
            vout[pl.ds(off, 16)] = val
            return carry2

        lax.fori_loop(0, SC_CH * (E // 16), rowcol_body, 0)
        pltpu.sync_copy(vout, out_hbm.at[pl.ds(row0 * E, SC_CH * E)])
        return carry

    lax.fori_loop(0, rows_per_w // SC_CH, chunk_body, 0)


def _sc_call(m0f, m1f, m2f, m3f, avf, spf):
    mesh = plsc.VectorSubcoreMesh(core_axis_name="c", subcore_axis_name="s")
    kern = functools.partial(
        pl.kernel,
        mesh=mesh,
        out_type=jax.ShapeDtypeStruct((B_SC * E,), jnp.float32),
        scratch_types=[
            pltpu.VMEM((SC_CH, E), jnp.float32),
            pltpu.VMEM((SC_CH, E), jnp.float32),
            pltpu.VMEM((SC_CH, E), jnp.float32),
            pltpu.VMEM((SC_CH, E), jnp.float32),
            pltpu.VMEM((B_SC // SC_WORKERS * M,), jnp.float32),
            pltpu.VMEM((B_SC // SC_WORKERS * M,), jnp.float32),
            pltpu.VMEM((SC_CH * M * 16,), jnp.float32),
            pltpu.VMEM((SC_CH * E,), jnp.float32),
            pltpu.SemaphoreType.DMA,
        ],
    )(_sc_worker)
    return kern(m0f, m1f, m2f, m3f, avf, spf)


@jax.jit
def kernel(m0, m1, m2, m3, availabilities, selection_probabilities):
    out_tc = _tc_call(
        m0, m1, m2, m3, availabilities, selection_probabilities
    )
    out_sc = _sc_call(
        m0,
        m1,
        m2,
        m3,
        availabilities.reshape(-1),
        selection_probabilities.reshape(-1),
    )
    return jnp.concatenate([out_tc, out_sc.reshape(B_SC, E)], axis=0)


# speedup vs baseline: 1.0310x; 1.0310x over previous
import functools

import jax
import jax.numpy as jnp
import numpy as np
from jax import lax
from jax.experimental import pallas as pl
from jax.experimental.pallas import tpu as pltpu
from jax.experimental.pallas import tpu_sc as plsc

B = 16384
E = 512
M = 4

B_SC = 3584
B_TC = B - B_SC
SC_WORKERS = 32
SC_CH = 16

_TINY = np.float32(1.17549435e-38)
_ROTS = (13, 15, 26, 6, 17, 29, 16, 24)

_LOG2_POLY = (
    -0.008764015229918067, 0.11976667205066446, -0.7261527889916303,
    2.5703314856108475, -5.882795874749627, 9.127889180021223,
    -9.888683565729947, 8.104570518183051, -3.416161479893353,
)


def _rotl(v, r):
    return lax.shift_left(v, jnp.int32(r)) | lax.shift_right_logical(
        v, jnp.int32(32 - r)
    )


def _threefry2x32_hash(cnt):
    ks = (jnp.int32(0), jnp.int32(42), jnp.int32(0 ^ 42 ^ 0x1BD11BDA))

    x1 = cnt + ks[1]
    x0 = x1
    x1 = x0 ^ _rotl(x1, _ROTS[0])
    for r in _ROTS[1:4]:
        x0 = x0 + x1
        x1 = x0 ^ _rotl(x1, r)
    x0 = x0 + ks[1]
    x1 = x1 + ks[2] + jnp.int32(1)
    for group in range(1, 5):
        rots = _ROTS[0:4] if group % 2 == 0 else _ROTS[4:8]
        for r in rots:
            x0 = x0 + x1
            x1 = x0 ^ _rotl(x1, r)
        x0 = x0 + ks[(group + 1) % 3]
        x1 = x1 + ks[(group + 2) % 3] + jnp.int32(group + 1)
    return x0 ^ x1


def _mantissa_bits(cnt):
    bits = _threefry2x32_hash(cnt)
    return lax.shift_right_logical(bits, jnp.int32(9))


def _tc_log2u(cnt):
    k = _mantissa_bits(cnt)
    fb = k | jnp.int32(0x3F800000)
    fl = lax.bitcast_convert_type(fb, jnp.float32) - jnp.float32(1.0)
    u = fl + _TINY
    return jnp.log2(u)


def _tc_body(m0_ref, m1_ref, m2_ref, m3_ref, av_ref, sp_ref, out_ref, *, tr, cr):
    i = pl.program_id(0)
    refs = (m0_ref, m1_ref, m2_ref, m3_ref)
    row = lax.broadcasted_iota(jnp.int32, (cr, E), 0)
    col = lax.broadcasted_iota(jnp.int32, (cr, E), 1)
    f_local = row * (E * M) + col * M

    def chunk_body(c, carry):
        r0 = c * cr
        f = (i * tr + r0) * (E * M) + f_local
        q = av_ref[pl.ds(r0, cr), :] * sp_ref[pl.ds(r0, cr), :]
        rq = jnp.float32(1.0) / q
        best = _tc_log2u(f) * rq[:, 0:1]
        val = refs[0][pl.ds(r0, cr), :]
        for m in range(1, M):
            s = _tc_log2u(f + m) * rq[:, m : m + 1]
            take = s > best
            val = jnp.where(take, refs[m][pl.ds(r0, cr), :], val)
            best = jnp.where(take, s, best)
        out_ref[pl.ds(r0, cr), :] = val
        return carry

    lax.fori_loop(0, tr // cr, chunk_body, 0, unroll=8)


def _tc_call(m0, m1, m2, m3, availabilities, selection_probabilities):
    tr = 512
    cr = 8
    grid = (B_TC // tr,)
    row_spec = pl.BlockSpec((tr, E), lambda i: (i, 0))
    prob_spec = pl.BlockSpec((tr, M), lambda i: (i, 0))
    return pl.pallas_call(
        functools.partial(_tc_body, tr=tr, cr=cr),
        grid=grid,
        in_specs=[row_spec, row_spec, row_spec, row_spec, prob_spec, prob_spec],
        out_specs=row_spec,
        out_shape=jax.ShapeDtypeStruct((B, E), jnp.float32),
        compiler_params=pltpu.CompilerParams(
            dimension_semantics=("parallel",)
        ),
    )(m0, m1, m2, m3, availabilities, selection_probabilities)


def _sc_log2u(cnt):
    k = _mantissa_bits(cnt)
    g = lax.convert_element_type(k, jnp.float32)
    gb = lax.bitcast_convert_type(g, jnp.int32)
    e = lax.convert_element_type(
        lax.shift_right_logical(gb, jnp.int32(23)) - jnp.int32(127),
        jnp.float32,
    )
    mant = lax.bitcast_convert_type(
        (gb & jnp.int32(0x007FFFFF)) | jnp.int32(0x3F800000), jnp.float32
    )
    p = jnp.float32(_LOG2_POLY[0])
    for c in _LOG2_POLY[1:]:
        p = p * mant + jnp.float32(c)
    return jnp.where(
        k == 0, jnp.float32(-126.0), (e + p) - jnp.float32(23.0)
    )


def _sc_worker(
    m0_hbm, m1_hbm, m2_hbm, m3_hbm, av_hbm, sp_hbm, out_hbm,
    vm0, vm1, vm2, vm3, vav, vsp, vrq, vout, sem,
):
    rows_per_w = B_SC // SC_WORKERS
    wid = lax.axis_index("s") * 2 + lax.axis_index("c")
    row_base = wid * rows_per_w
    vms = (vm0, vm1, vm2, vm3)
    mhbms = (m0_hbm, m1_hbm, m2_hbm, m3_hbm)
    lanes = lax.iota(jnp.int32, 16)

    pltpu.sync_copy(
        av_hbm.at[pl.ds((B_TC + row_base) * M, rows_per_w * M)], vav
    )
    pltpu.sync_copy(
        sp_hbm.at[pl.ds((B_TC + row_base) * M, rows_per_w * M)], vsp
    )

    def chunk_body(ch, carry):
        row0 = row_base + ch * SC_CH
        grow0 = B_TC + row0
        handles = [
            pltpu.async_copy(mh.at[pl.ds(grow0, SC_CH)], vm, sem)
            for vm, mh in zip(vms, mhbms)
        ]

        for r4 in range(SC_CH // 4):
            off4 = ch * (SC_CH * M) + r4 * 16
            q = vav[pl.ds(off4, 16)] * vsp[pl.ds(off4, 16)]
            rq = jnp.float32(1.0) / q
            for j in range(16):
                vrq[pl.ds((r4 * 16 + j) * 16, 16)] = jnp.full(
                    (16,), rq[j], jnp.float32
                )

        for h in handles:
            h.wait()

        def rowcol_body(t, carry2):
            r = t // (E // 16)
            g = lax.rem(t, E // 16)
            off = r * E + g * 16
            f0 = (grow0 + r) * (E * M) + g * (16 * M) + lanes * M
            c0 = g * 16
            best = _sc_log2u(f0) * vrq[pl.ds((r * 4) * 16, 16)]
            val = vm0[r, pl.ds(c0, 16)]
            for m in range(1, M):
                s = _sc_log2u(f0 + m) * vrq[pl.ds((r * 4 + m) * 16, 16)]
                take = s > best
                val = jnp.where(take, vms[m][r, pl.ds(c0, 16)], val)
                best = jnp.where(take, s, best)
            vout[pl.ds(off, 16)] = val
            return carry2

        lax.fori_loop(0, SC_CH * (E // 16), rowcol_body, 0)
        pltpu.sync_copy(vout, out_hbm.at[pl.ds(row0 * E, SC_CH * E)])
        return carry

    lax.fori_loop(0, rows_per_w // SC_CH, chunk_body, 0)


def _sc_call(m0f, m1f, m2f, m3f, avf, spf):
    mesh = plsc.VectorSubcoreMesh(core_axis_name="c", subcore_axis_name="s")
    kern = functools.partial(
        pl.kernel,
        mesh=mesh,
        out_type=jax.ShapeDtypeStruct((B_SC * E,), jnp.float32),
        scratch_types=[
            pltpu.VMEM((SC_CH, E), jnp.float32),
            pltpu.VMEM((SC_CH, E), jnp.float32),
            pltpu.VMEM((SC_CH, E), jnp.float32),
            pltpu.VMEM((SC_CH, E), jnp.float32),
            pltpu.VMEM((B_SC // SC_WORKERS * M,), jnp.float32),
            pltpu.VMEM((B_SC // SC_WORKERS * M,), jnp.float32),
            pltpu.VMEM((SC_CH * M * 16,), jnp.float32),
            pltpu.VMEM((SC_CH * E,), jnp.float32),
            pltpu.SemaphoreType.DMA,
        ],
    )(_sc_worker)
    return kern(m0f, m1f, m2f, m3f, avf, spf)


@jax.jit
def kernel(m0, m1, m2, m3, availabilities, selection_probabilities):
    out_tc = _tc_call(
        m0, m1, m2, m3, availabilities, selection_probabilities
    )
    out_sc = _sc_call(
        m0,
        m1,
        m2,
        m3,
        availabilities.reshape(-1),
        selection_probabilities.reshape(-1),
    )
    return lax.dynamic_update_slice(
        out_tc, out_sc.reshape(B_SC, E), (B_TC, 0)
    )
